# baseline (speedup 1.0000x reference)
import functools

import jax
import jax.numpy as jnp
from jax import lax
from jax.experimental import pallas as pl
from jax.experimental.pallas import tpu as pltpu
from jax.experimental.pallas import tpu_sc as plsc

B = 8
C = 64
N = 2048
K = 20
KP = 24
TN = 256
NT = N // TN
E = B * N * K
F32 = jnp.float32
EPS = 1e-5
BIG = 1e30
PREC = lax.Precision.DEFAULT


def _uv_body(x_ref, w1_ref, u_ref, v_ref):
    xb = x_ref[0]
    w1a = w1_ref[:, :C]
    w1d = w1_ref[:, C:] - w1a
    dn = (((0,), (1,)), ((), ()))
    u = lax.dot_general(xb, w1a, dn, precision=PREC,
                        preferred_element_type=F32)
    u_ref[:, :C] = u
    u_ref[:, C:] = u * u
    v_ref[...] = lax.dot_general(xb, w1d, dn, precision=PREC,
                                 preferred_element_type=F32)


def _run_uv(x, w1):
    return pl.pallas_call(
        _uv_body,
        grid=(B,),
        in_specs=[
            pl.BlockSpec((1, C, N), lambda b: (b, 0, 0)),
            pl.BlockSpec((C, 2 * C), lambda b: (0, 0)),
        ],
        out_specs=[
            pl.BlockSpec((N, 2 * C), lambda b: (b, 0)),
            pl.BlockSpec((N, C), lambda b: (b, 0)),
        ],
        out_shape=[
            jax.ShapeDtypeStruct((B * N, 2 * C), F32),
            jax.ShapeDtypeStruct((B * N, C), F32),
        ],
    )(x, w1)


SPLIT = 2
GB = B // SPLIT
TNK = 512


def _make_knn_body(b0):
    def _knn_body(x_ref, idx_ref):
        b = pl.program_id(0)
        nt = pl.program_id(1)
        xb = x_ref[0]
        xt = x_ref[0, :, pl.ds(nt * TNK, TNK)]
        xx = jnp.sum(xb * xb, axis=0, keepdims=True)
        ip = lax.dot_general(xt, xb, (((0,), (0,)), ((), ())),
                             precision=PREC, preferred_element_type=F32)
        pd = 2.0 * ip - xx
        base = (b + b0) * N
        iota = lax.broadcasted_iota(jnp.int32, (TNK, N), 1)
        am = None
        for t in range(K):
            am = jnp.argmax(pd, axis=1)
            idx_ref[0, t, :] = am + base
            pd = jnp.where(iota == am[:, None], -BIG, pd)
        pad = am + base
        for t in range(K, KP):
            idx_ref[0, t, :] = pad
    return _knn_body


def _run_knn(xg, b0):
    return pl.pallas_call(
        _make_knn_body(b0),
        grid=(GB, N // TNK),
        in_specs=[pl.BlockSpec((1, C, N), lambda b, nt: (b, 0, 0))],
        out_specs=pl.BlockSpec((1, KP, TNK), lambda b, nt: (b, 0, nt)),
        out_shape=jax.ShapeDtypeStruct((GB, KP, N), jnp.int32),
    )(xg)


EG = GB * N * K
NWIN = EG // 128
NWORK = 32
WPW = NWIN // NWORK


def _gather_body(table, idxw, out, idx0, idx1, rows0, rows1, sem0, sem1):
    wid = lax.axis_index("s") * 2 + lax.axis_index("c")
    base = wid * WPW
    idx_v = (idx0, idx1)
    rows_v = (rows0, rows1)
    sems = (sem0, sem1)

    for s in range(2):
        pltpu.sync_copy(idxw.at[base + s], idx_v[s])
        pltpu.async_copy(table.at[idx_v[s]], rows_v[s], sems[s])

    def outer(o, _):
        for s in range(2):
            i = o * 2 + s
            g = base + i
            pltpu.make_async_copy(table.at[idx_v[s]], rows_v[s], sems[s]).wait()
            pltpu.sync_copy(rows_v[s], out.at[pl.ds(g * 128, 128)])

            @pl.when(i + 2 < WPW)
            def _():
                pltpu.sync_copy(idxw.at[g + 2], idx_v[s])
                pltpu.async_copy(table.at[idx_v[s]], rows_v[s], sems[s])
        return _

    lax.fori_loop(0, WPW // 2, outer, None)


def _run_gather(u_table, idxw):
    mesh = plsc.VectorSubcoreMesh(core_axis_name="c", subcore_axis_name="s")
    f = functools.partial(
        pl.kernel,
        out_type=jax.ShapeDtypeStruct((EG, 2 * C), F32),
        mesh=mesh,
        scratch_types=[
            pltpu.VMEM((128,), jnp.int32),
            pltpu.VMEM((128,), jnp.int32),
            pltpu.VMEM((128, 2 * C), F32),
            pltpu.VMEM((128, 2 * C), F32),
            pltpu.SemaphoreType.DMA,
            pltpu.SemaphoreType.DMA,
        ],
    )(_gather_body)
    return f(u_table, idxw)


def _stats1_body(e_ref, v_ref, stats_ref, acc):
    step = pl.program_id(0) * NT + pl.program_id(1)

    @pl.when(step == 0)
    def _():
        acc[...] = jnp.zeros_like(acc)

    e = e_ref[0]
    vt = v_ref[...]
    s1 = jnp.sum(e[:, :, :C], axis=0)
    acc[0:1, :] += jnp.sum(s1, axis=0, keepdims=True)
    acc[1:2, :] += jnp.sum(jnp.sum(e[:, :, C:], axis=0), axis=0, keepdims=True)
    acc[2:3, :] += jnp.sum(vt * s1, axis=0, keepdims=True)
    acc[3:4, :] += jnp.sum(vt, axis=0, keepdims=True)
    acc[4:5, :] += jnp.sum(vt * vt, axis=0, keepdims=True)

    @pl.when(step == GB * NT - 1)
    def _():
        stats_ref[...] = acc[...]


def _run_stats1(e1, v_table):
    return pl.pallas_call(
        _stats1_body,
        grid=(GB, NT),
        in_specs=[
            pl.BlockSpec((1, K, TN, 2 * C), lambda b, nt: (b, 0, nt, 0)),
            pl.BlockSpec((TN, C), lambda b, nt: (b * NT + nt, 0)),
        ],
        out_specs=pl.BlockSpec((8, C), lambda b, nt: (0, 0)),
        out_shape=jax.ShapeDtypeStruct((8, C), F32),
        scratch_shapes=[pltpu.VMEM((8, C), F32)],
    )(e1, v_table)


def _conv2_body(e_ref, v_ref, s1_ref, w2_ref, g1_ref, b1_ref,
                mx_ref, mn_ref, stats2_ref, acc):
    step = pl.program_id(0) * NT + pl.program_id(1)

    @pl.when(step == 0)
    def _():
        acc[...] = jnp.zeros_like(acc)

    st = s1_ref[...]
    ecnt = F32(E)
    mean1 = (st[0:1] + F32(K) * st[3:4]) / ecnt
    ey2 = (st[1:2] + 2.0 * st[2:3] + F32(K) * st[4:5]) / ecnt
    var1 = ey2 - mean1 * mean1
    a1 = g1_ref[...] * lax.rsqrt(var1 + EPS)
    c1 = b1_ref[...] - mean1 * a1

    e = e_ref[0, :, :, :C]
    vt = v_ref[...]
    z = a1[None] * (e + vt[None]) + c1[None]
    z = jnp.where(z >= 0, z, 0.2 * z)
    zf = z.reshape(K * TN, C)
    y2 = lax.dot_general(zf, w2_ref[...], (((1,), (1,)), ((), ())),
                         precision=PREC, preferred_element_type=F32)
    y23 = y2.reshape(K, TN, 2 * C)
    mx_ref[0] = jnp.max(y23, axis=0)
    mn_ref[0] = jnp.min(y23, axis=0)
    acc[0:1, :] += jnp.sum(y2, axis=0, keepdims=True)
    acc[1:2, :] += jnp.sum(y2 * y2, axis=0, keepdims=True)

    @pl.when(step == GB * NT - 1)
    def _():
        stats2_ref[...] = acc[...]


def _run_conv2(e1, v_table, stats1, w2, g1r, b1r):
    return pl.pallas_call(
        _conv2_body,
        grid=(GB, NT),
        in_specs=[
            pl.BlockSpec((1, K, TN, 2 * C), lambda b, nt: (b, 0, nt, 0)),
            pl.BlockSpec((TN, C), lambda b, nt: (b * NT + nt, 0)),
            pl.BlockSpec((8, C), lambda b, nt: (0, 0)),
            pl.BlockSpec((2 * C, C), lambda b, nt: (0, 0)),
            pl.BlockSpec((1, C), lambda b, nt: (0, 0)),
            pl.BlockSpec((1, C), lambda b, nt: (0, 0)),
        ],
        out_specs=[
            pl.BlockSpec((1, TN, 2 * C), lambda b, nt: (b, nt, 0)),
            pl.BlockSpec((1, TN, 2 * C), lambda b, nt: (b, nt, 0)),
            pl.BlockSpec((8, 2 * C), lambda b, nt: (0, 0)),
        ],
        out_shape=[
            jax.ShapeDtypeStruct((GB, N, 2 * C), F32),
            jax.ShapeDtypeStruct((GB, N, 2 * C), F32),
            jax.ShapeDtypeStruct((8, 2 * C), F32),
        ],
        scratch_shapes=[pltpu.VMEM((8, 2 * C), F32)],
    )(e1, v_table, stats1, w2, g1r, b1r)


def _final_body(mx_ref, mn_ref, s2_ref, g2_ref, b2_ref, out_ref):
    st = s2_ref[...]
    ecnt = F32(E)
    mean2 = st[0:1] / ecnt
    var2 = st[1:2] / ecnt - mean2 * mean2
    a2 = g2_ref[...] * lax.rsqrt(var2 + EPS)
    c2 = b2_ref[...] - mean2 * a2
    m = jnp.where(a2 >= 0, mx_ref[0], mn_ref[0])
    y = a2 * m + c2
    out_ref[0] = jnp.where(y >= 0, y, 0.2 * y)


def _run_final(mx, mn, stats2, g2r, b2r):
    return pl.pallas_call(
        _final_body,
        grid=(GB, NT),
        in_specs=[
            pl.BlockSpec((1, TN, 2 * C), lambda b, nt: (b, nt, 0)),
            pl.BlockSpec((1, TN, 2 * C), lambda b, nt: (b, nt, 0)),
            pl.BlockSpec((8, 2 * C), lambda b, nt: (0, 0)),
            pl.BlockSpec((1, 2 * C), lambda b, nt: (0, 0)),
            pl.BlockSpec((1, 2 * C), lambda b, nt: (0, 0)),
        ],
        out_specs=pl.BlockSpec((1, TN, 2 * C), lambda b, nt: (b, nt, 0)),
        out_shape=jax.ShapeDtypeStruct((GB, N, 2 * C), F32),
    )(mx, mn, stats2, g2r, b2r)


def kernel(x, W1, W2, g1, b1, g2, b2):
    u_table, v_table = _run_uv(x, W1)
    e1_parts, v_parts = [], []
    for gi in range(SPLIT):
        xg = x[gi * GB:(gi + 1) * GB]
        idxp = _run_knn(xg, gi * GB)
        idxw = idxp[:, :K, :].reshape(NWIN, 128)
        e1_parts.append(_run_gather(u_table, idxw).reshape(GB, K, N, 2 * C))
        v_parts.append(v_table[gi * GB * N:(gi + 1) * GB * N])
    stats1 = _run_stats1(e1_parts[0], v_parts[0])
    for gi in range(1, SPLIT):
        stats1 = stats1 + _run_stats1(e1_parts[gi], v_parts[gi])
    g1r, b1r = g1.reshape(1, C), b1.reshape(1, C)
    outs, stats2 = [], None
    parts2 = []
    for gi in range(SPLIT):
        mx, mn, s2 = _run_conv2(e1_parts[gi], v_parts[gi], stats1, W2,
                                g1r, b1r)
        parts2.append((mx, mn))
        stats2 = s2 if stats2 is None else stats2 + s2
    g2r, b2r = g2.reshape(1, 2 * C), b2.reshape(1, 2 * C)
    for gi in range(SPLIT):
        mx, mn = parts2[gi]
        outs.append(_run_final(mx, mn, stats2, g2r, b2r))
    out_t = jnp.concatenate(outs, axis=0)
    return jnp.transpose(out_t, (0, 2, 1))


# speedup vs baseline: 1.0780x; 1.0780x over previous
import functools

import jax
import jax.numpy as jnp
from jax import lax
from jax.experimental import pallas as pl
from jax.experimental.pallas import tpu as pltpu
from jax.experimental.pallas import tpu_sc as plsc

B = 8
C = 64
N = 2048
K = 20
KP = 24
TN = 256
NT = N // TN
E = B * N * K
F32 = jnp.float32
EPS = 1e-5
BIG = 1e30
PREC = lax.Precision.DEFAULT


def _uv_body(x_ref, w1_ref, u_ref, v_ref):
    xb = x_ref[0]
    w1a = w1_ref[:, :C]
    w1d = w1_ref[:, C:] - w1a
    dn = (((0,), (1,)), ((), ()))
    u = lax.dot_general(xb, w1a, dn, precision=PREC,
                        preferred_element_type=F32)
    u_ref[:, :C] = u
    u_ref[:, C:] = u * u
    v_ref[...] = lax.dot_general(xb, w1d, dn, precision=PREC,
                                 preferred_element_type=F32)


def _run_uv(x, w1):
    return pl.pallas_call(
        _uv_body,
        grid=(B,),
        in_specs=[
            pl.BlockSpec((1, C, N), lambda b: (b, 0, 0)),
            pl.BlockSpec((C, 2 * C), lambda b: (0, 0)),
        ],
        out_specs=[
            pl.BlockSpec((N, 2 * C), lambda b: (b, 0)),
            pl.BlockSpec((N, C), lambda b: (b, 0)),
        ],
        out_shape=[
            jax.ShapeDtypeStruct((B * N, 2 * C), F32),
            jax.ShapeDtypeStruct((B * N, C), F32),
        ],
    )(x, w1)


SPLIT = 4
GB = B // SPLIT
TNK = 256


def _make_knn_body(b0):
    def _knn_body(x_ref, idx_ref):
        b = pl.program_id(0)
        nt = pl.program_id(1)
        xb = x_ref[0]
        xt = x_ref[0, :, pl.ds(nt * TNK, TNK)]
        xx = jnp.sum(xb * xb, axis=0, keepdims=True)
        ip = lax.dot_general(xt, xb, (((0,), (0,)), ((), ())),
                             precision=PREC, preferred_element_type=F32)
        pd = 2.0 * ip - xx
        base = (b + b0) * N
        iota = lax.broadcasted_iota(jnp.int32, (TNK, N), 1)
        am = None
        for t in range(K):
            am = jnp.argmax(pd, axis=1)
            idx_ref[0, t, :] = am + base
            pd = jnp.where(iota == am[:, None], -BIG, pd)
        pad = am + base
        for t in range(K, KP):
            idx_ref[0, t, :] = pad
    return _knn_body


def _run_knn(xg, b0):
    return pl.pallas_call(
        _make_knn_body(b0),
        grid=(GB, N // TNK),
        in_specs=[pl.BlockSpec((1, C, N), lambda b, nt: (b, 0, 0))],
        out_specs=pl.BlockSpec((1, KP, TNK), lambda b, nt: (b, 0, nt)),
        out_shape=jax.ShapeDtypeStruct((GB, KP, N), jnp.int32),
    )(xg)


EG = GB * N * K
NWIN = EG // 128
NWORK = 32
WPW = NWIN // NWORK


def _gather_body(table, idxw, out, idx0, idx1, rows0, rows1, sem0, sem1):
    wid = lax.axis_index("s") * 2 + lax.axis_index("c")
    base = wid * WPW
    idx_v = (idx0, idx1)
    rows_v = (rows0, rows1)
    sems = (sem0, sem1)

    for s in range(2):
        pltpu.sync_copy(idxw.at[base + s], idx_v[s])
        pltpu.async_copy(table.at[idx_v[s]], rows_v[s], sems[s])

    def outer(o, _):
        for s in range(2):
            i = o * 2 + s
            g = base + i
            pltpu.make_async_copy(table.at[idx_v[s]], rows_v[s], sems[s]).wait()
            pltpu.sync_copy(rows_v[s], out.at[pl.ds(g * 128, 128)])

            @pl.when(i + 2 < WPW)
            def _():
                pltpu.sync_copy(idxw.at[g + 2], idx_v[s])
                pltpu.async_copy(table.at[idx_v[s]], rows_v[s], sems[s])
        return _

    lax.fori_loop(0, WPW // 2, outer, None)


def _run_gather(u_table, idxw):
    mesh = plsc.VectorSubcoreMesh(core_axis_name="c", subcore_axis_name="s")
    f = functools.partial(
        pl.kernel,
        out_type=jax.ShapeDtypeStruct((EG, 2 * C), F32),
        mesh=mesh,
        scratch_types=[
            pltpu.VMEM((128,), jnp.int32),
            pltpu.VMEM((128,), jnp.int32),
            pltpu.VMEM((128, 2 * C), F32),
            pltpu.VMEM((128, 2 * C), F32),
            pltpu.SemaphoreType.DMA,
            pltpu.SemaphoreType.DMA,
        ],
    )(_gather_body)
    return f(u_table, idxw)


def _stats1_body(e_ref, v_ref, stats_ref, acc):
    step = pl.program_id(0) * NT + pl.program_id(1)

    @pl.when(step == 0)
    def _():
        acc[...] = jnp.zeros_like(acc)

    e = e_ref[0]
    vt = v_ref[...]
    s1 = jnp.sum(e[:, :, :C], axis=0)
    acc[0:1, :] += jnp.sum(s1, axis=0, keepdims=True)
    acc[1:2, :] += jnp.sum(jnp.sum(e[:, :, C:], axis=0), axis=0, keepdims=True)
    acc[2:3, :] += jnp.sum(vt * s1, axis=0, keepdims=True)
    acc[3:4, :] += jnp.sum(vt, axis=0, keepdims=True)
    acc[4:5, :] += jnp.sum(vt * vt, axis=0, keepdims=True)

    @pl.when(step == GB * NT - 1)
    def _():
        stats_ref[...] = acc[...]


def _run_stats1(e1, v_table):
    return pl.pallas_call(
        _stats1_body,
        grid=(GB, NT),
        in_specs=[
            pl.BlockSpec((1, K, TN, 2 * C), lambda b, nt: (b, 0, nt, 0)),
            pl.BlockSpec((TN, C), lambda b, nt: (b * NT + nt, 0)),
        ],
        out_specs=pl.BlockSpec((8, C), lambda b, nt: (0, 0)),
        out_shape=jax.ShapeDtypeStruct((8, C), F32),
        scratch_shapes=[pltpu.VMEM((8, C), F32)],
    )(e1, v_table)


def _conv2_body(e_ref, v_ref, s1_ref, w2_ref, g1_ref, b1_ref,
                mx_ref, mn_ref, stats2_ref, acc):
    step = pl.program_id(0) * NT + pl.program_id(1)

    @pl.when(step == 0)
    def _():
        acc[...] = jnp.zeros_like(acc)

    st = s1_ref[...]
    ecnt = F32(E)
    mean1 = (st[0:1] + F32(K) * st[3:4]) / ecnt
    ey2 = (st[1:2] + 2.0 * st[2:3] + F32(K) * st[4:5]) / ecnt
    var1 = ey2 - mean1 * mean1
    a1 = g1_ref[...] * lax.rsqrt(var1 + EPS)
    c1 = b1_ref[...] - mean1 * a1

    e = e_ref[0, :, :, :C]
    vt = v_ref[...]
    z = a1[None] * (e + vt[None]) + c1[None]
    z = jnp.where(z >= 0, z, 0.2 * z)
    zf = z.reshape(K * TN, C)
    y2 = lax.dot_general(zf, w2_ref[...], (((1,), (1,)), ((), ())),
                         precision=PREC, preferred_element_type=F32)
    y23 = y2.reshape(K, TN, 2 * C)
    mx_ref[0] = jnp.max(y23, axis=0)
    mn_ref[0] = jnp.min(y23, axis=0)
    acc[0:1, :] += jnp.sum(y2, axis=0, keepdims=True)
    acc[1:2, :] += jnp.sum(y2 * y2, axis=0, keepdims=True)

    @pl.when(step == GB * NT - 1)
    def _():
        stats2_ref[...] = acc[...]


def _run_conv2(e1, v_table, stats1, w2, g1r, b1r):
    return pl.pallas_call(
        _conv2_body,
        grid=(GB, NT),
        in_specs=[
            pl.BlockSpec((1, K, TN, 2 * C), lambda b, nt: (b, 0, nt, 0)),
            pl.BlockSpec((TN, C), lambda b, nt: (b * NT + nt, 0)),
            pl.BlockSpec((8, C), lambda b, nt: (0, 0)),
            pl.BlockSpec((2 * C, C), lambda b, nt: (0, 0)),
            pl.BlockSpec((1, C), lambda b, nt: (0, 0)),
            pl.BlockSpec((1, C), lambda b, nt: (0, 0)),
        ],
        out_specs=[
            pl.BlockSpec((1, TN, 2 * C), lambda b, nt: (b, nt, 0)),
            pl.BlockSpec((1, TN, 2 * C), lambda b, nt: (b, nt, 0)),
            pl.BlockSpec((8, 2 * C), lambda b, nt: (0, 0)),
        ],
        out_shape=[
            jax.ShapeDtypeStruct((GB, N, 2 * C), F32),
            jax.ShapeDtypeStruct((GB, N, 2 * C), F32),
            jax.ShapeDtypeStruct((8, 2 * C), F32),
        ],
        scratch_shapes=[pltpu.VMEM((8, 2 * C), F32)],
    )(e1, v_table, stats1, w2, g1r, b1r)


def _final_body(mx_ref, mn_ref, s2_ref, g2_ref, b2_ref, out_ref):
    st = s2_ref[...]
    ecnt = F32(E)
    mean2 = st[0:1] / ecnt
    var2 = st[1:2] / ecnt - mean2 * mean2
    a2 = g2_ref[...] * lax.rsqrt(var2 + EPS)
    c2 = b2_ref[...] - mean2 * a2
    m = jnp.where(a2 >= 0, mx_ref[0], mn_ref[0])
    y = a2 * m + c2
    out_ref[0] = jnp.where(y >= 0, y, 0.2 * y)


def _run_final(mx, mn, stats2, g2r, b2r):
    return pl.pallas_call(
        _final_body,
        grid=(GB, NT),
        in_specs=[
            pl.BlockSpec((1, TN, 2 * C), lambda b, nt: (b, nt, 0)),
            pl.BlockSpec((1, TN, 2 * C), lambda b, nt: (b, nt, 0)),
            pl.BlockSpec((8, 2 * C), lambda b, nt: (0, 0)),
            pl.BlockSpec((1, 2 * C), lambda b, nt: (0, 0)),
            pl.BlockSpec((1, 2 * C), lambda b, nt: (0, 0)),
        ],
        out_specs=pl.BlockSpec((1, TN, 2 * C), lambda b, nt: (b, nt, 0)),
        out_shape=jax.ShapeDtypeStruct((GB, N, 2 * C), F32),
    )(mx, mn, stats2, g2r, b2r)


def kernel(x, W1, W2, g1, b1, g2, b2):
    u_table, v_table = _run_uv(x, W1)
    e1_parts, v_parts = [], []
    for gi in range(SPLIT):
        xg = x[gi * GB:(gi + 1) * GB]
        idxp = _run_knn(xg, gi * GB)
        idxw = idxp[:, :K, :].reshape(NWIN, 128)
        e1_parts.append(_run_gather(u_table, idxw).reshape(GB, K, N, 2 * C))
        v_parts.append(v_table[gi * GB * N:(gi + 1) * GB * N])
    stats1 = _run_stats1(e1_parts[0], v_parts[0])
    for gi in range(1, SPLIT):
        stats1 = stats1 + _run_stats1(e1_parts[gi], v_parts[gi])
    g1r, b1r = g1.reshape(1, C), b1.reshape(1, C)
    outs, stats2 = [], None
    parts2 = []
    for gi in range(SPLIT):
        mx, mn, s2 = _run_conv2(e1_parts[gi], v_parts[gi], stats1, W2,
                                g1r, b1r)
        parts2.append((mx, mn))
        stats2 = s2 if stats2 is None else stats2 + s2
    g2r, b2r = g2.reshape(1, 2 * C), b2.reshape(1, 2 * C)
    for gi in range(SPLIT):
        mx, mn = parts2[gi]
        outs.append(_run_final(mx, mn, stats2, g2r, b2r))
    out_t = jnp.concatenate(outs, axis=0)
    return jnp.transpose(out_t, (0, 2, 1))
